# baseline (speedup 1.0000x reference)
import functools

import jax
import jax.numpy as jnp
import numpy as np
from jax import lax
from jax.experimental import pallas as pl
from jax.experimental.pallas import tpu as pltpu
from jax.experimental.pallas import tpu_sc as plsc

EPS = 1e-15
MAX_NORM = 1.0 - 1e-5
ATANH_MAX = float(np.arctanh(np.float64(MAX_NORM)))
N_NODES = 10000
D = 256
HALF = 128
QW = 64
N_EDGES = 160000

NC = 2
NS = 16
EPT = N_EDGES // NS
K = 80
NCHUNK = EPT // K
NBUF = 5
NGRP = NCHUNK // NBUF
RPT = 624
TAIL0 = NS * RPT
TAIL = N_NODES - TAIL0
ZR = 208

DEPT = N_EDGES // (NC * NS)
DK = 100
DNCHUNK = DEPT // DK

BN = 5000
GRID = N_NODES // BN


def _norm(x):
    return jnp.sqrt(jnp.sum(x * x, axis=-1, keepdims=True))


def _log_exp_map0(u):
    n = _norm(u)
    return u * (jnp.minimum(n, ATANH_MAX) / jnp.maximum(n, EPS))


def _expmap0(u):
    norm = _norm(u)
    safe = jnp.maximum(norm, EPS)
    y = jnp.tanh(norm) * u / safe
    ynorm = _norm(y)
    scaled = y / jnp.maximum(ynorm, EPS) * MAX_NORM
    return jnp.where(ynorm > MAX_NORM, scaled, y)


def _dot(t, w):
    return jnp.dot(t, w, precision=lax.Precision.HIGHEST,
                   preferred_element_type=jnp.float32)


def _tc_pre_body(x_ref, w_ref, b_ref, outl_ref, outr_ref):
    t = _log_exp_map0(0.1 * x_ref[...])
    h = _dot(t, w_ref[...]) + b_ref[...]
    outl_ref[...] = h[:, :HALF]
    outr_ref[...] = h[:, HALF:]


def _tc_pre(x, W, b):
    return pl.pallas_call(
        _tc_pre_body,
        grid=(GRID,),
        in_specs=[pl.BlockSpec((BN, D), lambda i: (i, 0)),
                  pl.BlockSpec((D, D), lambda i: (0, 0)),
                  pl.BlockSpec((1, D), lambda i: (0, 0))],
        out_specs=[pl.BlockSpec((BN, HALF), lambda i: (i, 0)),
                   pl.BlockSpec((BN, HALF), lambda i: (i, 0))],
        out_shape=[jax.ShapeDtypeStruct((N_NODES, HALF), jnp.float32),
                   jax.ShapeDtypeStruct((N_NODES, HALF), jnp.float32)],
    )(x, W, b)


def _make_mid(use_relu):
    def body(al_ref, ar_ref, dega_ref, degb_ref, w_ref, b_ref,
             outl_ref, outr_ref):
        agg = jnp.concatenate([al_ref[...], ar_ref[...]], axis=-1)
        if use_relu:
            agg = jnp.maximum(agg, 0.0)
        deg = jnp.maximum((dega_ref[...] + degb_ref[...])[:, :1], 1.0)
        na = _norm(agg)
        nu = na / deg
        factor = jnp.minimum(nu, ATANH_MAX) / jnp.maximum(nu, EPS) / deg
        t = agg * factor
        h = _dot(t, w_ref[...]) + b_ref[...]
        outl_ref[...] = h[:, :HALF]
        outr_ref[...] = h[:, HALF:]

    def call(al, ar, dega, degb, W, b):
        return pl.pallas_call(
            body,
            grid=(GRID,),
            in_specs=[pl.BlockSpec((BN, HALF), lambda i: (i, 0)),
                      pl.BlockSpec((BN, HALF), lambda i: (i, 0)),
                      pl.BlockSpec((BN, 16), lambda i: (i, 0)),
                      pl.BlockSpec((BN, 16), lambda i: (i, 0)),
                      pl.BlockSpec((D, D), lambda i: (0, 0)),
                      pl.BlockSpec((1, D), lambda i: (0, 0))],
            out_specs=[pl.BlockSpec((BN, HALF), lambda i: (i, 0)),
                       pl.BlockSpec((BN, HALF), lambda i: (i, 0))],
            out_shape=[jax.ShapeDtypeStruct((N_NODES, HALF), jnp.float32),
                       jax.ShapeDtypeStruct((N_NODES, HALF), jnp.float32)],
        )(al, ar, dega, degb, W, b)

    return call


_tc_mid_noact = _make_mid(False)
_tc_mid_act = _make_mid(True)


def _tc_out_body(al_ref, ar_ref, dega_ref, degb_ref, out_ref):
    agg = jnp.maximum(
        jnp.concatenate([al_ref[...], ar_ref[...]], axis=-1), 0.0)
    deg = jnp.maximum((dega_ref[...] + degb_ref[...])[:, :1], 1.0)
    nu = _norm(agg) / deg
    ty = jnp.tanh(nu)
    factor = ty / jnp.maximum(nu, EPS)
    factor = jnp.where(ty > MAX_NORM,
                       MAX_NORM / jnp.maximum(nu, EPS), factor)
    out_ref[...] = agg * (factor / deg)


def _tc_out(al, ar, dega, degb):
    return pl.pallas_call(
        _tc_out_body,
        grid=(GRID,),
        in_specs=[pl.BlockSpec((BN, HALF), lambda i: (i, 0)),
                  pl.BlockSpec((BN, HALF), lambda i: (i, 0)),
                  pl.BlockSpec((BN, 16), lambda i: (i, 0)),
                  pl.BlockSpec((BN, 16), lambda i: (i, 0))],
        out_specs=pl.BlockSpec((BN, D), lambda i: (i, 0)),
        out_shape=jax.ShapeDtypeStruct((N_NODES, D), jnp.float32),
    )(al, ar, dega, degb)


def _init_slab(tid, z_hbm, sh):
    for s in range(RPT // ZR):
        pltpu.sync_copy(z_hbm, sh.at[pl.ds(tid * RPT + s * ZR, ZR)])

    @pl.when(tid == 0)
    def _():
        pltpu.sync_copy(z_hbm.at[pl.ds(0, TAIL)], sh.at[pl.ds(TAIL0, TAIL)])


def _copy_out_slab(tid, sh, out, base):
    pltpu.sync_copy(sh.at[pl.ds(tid * RPT, RPT)],
                    out.at[pl.ds(base + tid * RPT, RPT)])

    @pl.when(tid == 0)
    def _():
        pltpu.sync_copy(sh.at[pl.ds(TAIL0, TAIL)],
                        out.at[pl.ds(base + TAIL0, TAIL)])


@functools.cache
def _sc_mesh():
    return plsc.VectorSubcoreMesh(core_axis_name="c", subcore_axis_name="s")


@functools.cache
def _sc_agg_kernel():
    return functools.partial(
        pl.kernel, mesh=_sc_mesh(),
        compiler_params=pltpu.CompilerParams(use_tc_tiling_on_sc=False),
        out_type=(jax.ShapeDtypeStruct((N_NODES, NC * QW), jnp.float32),
                  jax.ShapeDtypeStruct((N_NODES, NC * QW), jnp.float32)),
        scratch_types=[
            pltpu.VMEM((NCHUNK, K), jnp.int32),
            pltpu.VMEM((NCHUNK, K), jnp.int32),
            pltpu.VMEM((NBUF, K, QW), jnp.float32),
            pltpu.VMEM_SHARED((N_NODES, QW), jnp.float32),
            pltpu.SemaphoreType.DMA((NBUF,)),
            pltpu.SemaphoreType.DMA((NBUF,)),
        ])(_sc_agg_body)


def _sc_agg(hl2, hr2, src_all, dst_all, z64):
    return _sc_agg_kernel()(hl2, hr2, src_all, dst_all, z64)


def _sc_agg_body(hl_hbm, hr_hbm, src_hbm, dst_hbm, z64_hbm,
                 aggl_out, aggr_out,
                 src_v, dst_v, rows, agg_sh, gsem, ssem):
    cid = lax.axis_index("c")
    tid = lax.axis_index("s")
    wid = cid * NS + tid
    pltpu.sync_copy(src_hbm.at[wid], src_v)
    pltpu.sync_copy(dst_hbm.at[tid], dst_v)
    _init_slab(tid, z64_hbm, agg_sh)

    def copy_out(out):
        pltpu.sync_copy(agg_sh.at[pl.ds(tid * RPT, RPT)],
                        out.at[pl.ds(tid * RPT, RPT), pl.ds(QW * cid, QW)])

        @pl.when(tid == 0)
        def _():
            pltpu.sync_copy(agg_sh.at[pl.ds(TAIL0, TAIL)],
                            out.at[pl.ds(TAIL0, TAIL), pl.ds(QW * cid, QW)])

    def phase(h_hbm, out):
        plsc.subcore_barrier()

        def gather(j, b):
            pltpu.async_copy(h_hbm.at[src_v.at[j]], rows.at[b], gsem.at[b])

        def wait_gather(b):
            pltpu.make_async_copy(h_hbm.at[pl.ds(0, K)], rows.at[b],
                                  gsem.at[b]).wait()

        def wait_scatter(b):
            pltpu.make_async_copy(rows.at[b], agg_sh.at[pl.ds(0, K)],
                                  ssem.at[b]).wait()

        for b in range(NBUF):
            gather(b, b)

        def group_body(g, carry):
            for b in range(NBUF):
                j = g * NBUF + b
                wait_gather(b)
                pltpu.async_copy(rows.at[b], agg_sh.at[dst_v.at[j]],
                                 ssem.at[b], add=True)

                @pl.when(g < NGRP - 1)
                def _():
                    wait_scatter(b)
                    gather(j + NBUF, b)

            return carry

        lax.fori_loop(0, NGRP, group_body, 0)
        for b in range(NBUF):
            wait_scatter(b)

        plsc.subcore_barrier()
        copy_out(out)

    phase(hl_hbm, aggl_out)
    _init_slab(tid, z64_hbm, agg_sh)
    phase(hr_hbm, aggr_out)


@functools.cache
def _sc_deg_kernel():
    return functools.partial(
        pl.kernel, mesh=_sc_mesh(),
        compiler_params=pltpu.CompilerParams(use_tc_tiling_on_sc=False),
        out_type=(jax.ShapeDtypeStruct((N_NODES, 16), jnp.float32),
                  jax.ShapeDtypeStruct((N_NODES, 16), jnp.float32)),
        scratch_types=[
            pltpu.VMEM((DNCHUNK, DK), jnp.int32),
            pltpu.VMEM((DK, 16), jnp.float32),
            pltpu.VMEM_SHARED((N_NODES, 16), jnp.float32),
        ])(_sc_deg_body)


def _sc_deg(dstd, z16, ones16):
    return _sc_deg_kernel()(dstd, z16, ones16)


def _sc_deg_body(dst_hbm, z16_hbm, ones_hbm,
                 dega_out, degb_out,
                 dst_v, ones_v, deg_sh):
    cid = lax.axis_index("c")
    tid = lax.axis_index("s")
    wid = cid * NS + tid
    pltpu.sync_copy(dst_hbm.at[wid], dst_v)
    pltpu.sync_copy(ones_hbm, ones_v)
    _init_slab(tid, z16_hbm, deg_sh)
    plsc.subcore_barrier()

    def body(j, carry):
        pltpu.sync_copy(ones_v, deg_sh.at[dst_v.at[j]], add=True)
        return carry

    lax.fori_loop(0, DNCHUNK, body, 0)

    plsc.subcore_barrier()

    @pl.when(cid == 0)
    def _():
        _copy_out_slab(tid, deg_sh, dega_out, 0)

    @pl.when(cid == 1)
    def _():
        _copy_out_slab(tid, deg_sh, degb_out, 0)


def kernel(x, edge_index, W0, b0, W1, b1, W2, b2):
    src = edge_index[0].astype(jnp.int32)
    dst = edge_index[1].astype(jnp.int32)
    src2 = (2 * src).reshape(1, NS, NCHUNK, K)
    src_all = jnp.concatenate([src2, src2 + 1], axis=0)
    src_all = src_all.reshape(NC * NS, NCHUNK, K)
    dst_all = dst.reshape(NS, NCHUNK, K)
    dst_deg = dst.reshape(NC * NS, DNCHUNK, DK)
    z64 = jnp.zeros((ZR, QW), jnp.float32)
    z16 = jnp.zeros((ZR, 16), jnp.float32)
    ones16 = jnp.ones((DK, 16), jnp.float32)

    def agg_layer(tl, tr):
        return _sc_agg(tl.reshape(NC * N_NODES, QW),
                       tr.reshape(NC * N_NODES, QW), src_all, dst_all, z64)

    dega, degb = _sc_deg(dst_deg, z16, ones16)
    tl, tr = _tc_pre(x, W0, b0.reshape(1, D))
    al, ar = agg_layer(tl, tr)
    tl, tr = _tc_mid_noact(al, ar, dega, degb, W1, b1.reshape(1, D))
    al, ar = agg_layer(tl, tr)
    tl, tr = _tc_mid_act(al, ar, dega, degb, W2, b2.reshape(1, D))
    al, ar = agg_layer(tl, tr)
    return _tc_out(al, ar, dega, degb)


# speedup vs baseline: 1.0123x; 1.0123x over previous
import functools

import jax
import jax.numpy as jnp
import numpy as np
from jax import lax
from jax.experimental import pallas as pl
from jax.experimental.pallas import tpu as pltpu
from jax.experimental.pallas import tpu_sc as plsc

EPS = 1e-15
MAX_NORM = 1.0 - 1e-5
ATANH_MAX = float(np.arctanh(np.float64(MAX_NORM)))
N_NODES = 10000
D = 256
HALF = 128
QW = 64
N_EDGES = 160000

NC = 2
NS = 16
EPT = N_EDGES // NS
K = 80
NCHUNK = EPT // K
NBUF = 5
NGRP = NCHUNK // NBUF
RPT = 624
TAIL0 = NS * RPT
TAIL = N_NODES - TAIL0
ZR = 208

DEPT = N_EDGES // (NC * NS)
DK = 100
DNCHUNK = DEPT // DK

BN = 2000
GRID = N_NODES // BN


def _norm(x):
    return jnp.sqrt(jnp.sum(x * x, axis=-1, keepdims=True))


def _log_exp_map0(u):
    n = _norm(u)
    return u * (jnp.minimum(n, ATANH_MAX) / jnp.maximum(n, EPS))


def _expmap0(u):
    norm = _norm(u)
    safe = jnp.maximum(norm, EPS)
    y = jnp.tanh(norm) * u / safe
    ynorm = _norm(y)
    scaled = y / jnp.maximum(ynorm, EPS) * MAX_NORM
    return jnp.where(ynorm > MAX_NORM, scaled, y)


def _dot(t, w):
    return jnp.dot(t, w, precision=lax.Precision.HIGHEST,
                   preferred_element_type=jnp.float32)


def _tc_pre_body(x_ref, w_ref, b_ref, outl_ref, outr_ref):
    t = _log_exp_map0(0.1 * x_ref[...])
    h = _dot(t, w_ref[...]) + b_ref[...]
    outl_ref[...] = h[:, :HALF]
    outr_ref[...] = h[:, HALF:]


def _tc_pre(x, W, b):
    return pl.pallas_call(
        _tc_pre_body,
        grid=(GRID,),
        in_specs=[pl.BlockSpec((BN, D), lambda i: (i, 0)),
                  pl.BlockSpec((D, D), lambda i: (0, 0)),
                  pl.BlockSpec((1, D), lambda i: (0, 0))],
        out_specs=[pl.BlockSpec((BN, HALF), lambda i: (i, 0)),
                   pl.BlockSpec((BN, HALF), lambda i: (i, 0))],
        out_shape=[jax.ShapeDtypeStruct((N_NODES, HALF), jnp.float32),
                   jax.ShapeDtypeStruct((N_NODES, HALF), jnp.float32)],
    )(x, W, b)


def _make_mid(use_relu):
    def body(al_ref, ar_ref, dega_ref, degb_ref, w_ref, b_ref,
             outl_ref, outr_ref):
        agg = jnp.concatenate([al_ref[...], ar_ref[...]], axis=-1)
        if use_relu:
            agg = jnp.maximum(agg, 0.0)
        deg = jnp.maximum((dega_ref[...] + degb_ref[...])[:, :1], 1.0)
        na = _norm(agg)
        nu = na / deg
        factor = jnp.minimum(nu, ATANH_MAX) / jnp.maximum(nu, EPS) / deg
        t = agg * factor
        h = _dot(t, w_ref[...]) + b_ref[...]
        outl_ref[...] = h[:, :HALF]
        outr_ref[...] = h[:, HALF:]

    def call(al, ar, dega, degb, W, b):
        return pl.pallas_call(
            body,
            grid=(GRID,),
            in_specs=[pl.BlockSpec((BN, HALF), lambda i: (i, 0)),
                      pl.BlockSpec((BN, HALF), lambda i: (i, 0)),
                      pl.BlockSpec((BN, 16), lambda i: (i, 0)),
                      pl.BlockSpec((BN, 16), lambda i: (i, 0)),
                      pl.BlockSpec((D, D), lambda i: (0, 0)),
                      pl.BlockSpec((1, D), lambda i: (0, 0))],
            out_specs=[pl.BlockSpec((BN, HALF), lambda i: (i, 0)),
                       pl.BlockSpec((BN, HALF), lambda i: (i, 0))],
            out_shape=[jax.ShapeDtypeStruct((N_NODES, HALF), jnp.float32),
                       jax.ShapeDtypeStruct((N_NODES, HALF), jnp.float32)],
        )(al, ar, dega, degb, W, b)

    return call


_tc_mid_noact = _make_mid(False)
_tc_mid_act = _make_mid(True)


def _tc_out_body(al_ref, ar_ref, dega_ref, degb_ref, out_ref):
    agg = jnp.maximum(
        jnp.concatenate([al_ref[...], ar_ref[...]], axis=-1), 0.0)
    deg = jnp.maximum((dega_ref[...] + degb_ref[...])[:, :1], 1.0)
    nu = _norm(agg) / deg
    ty = jnp.tanh(nu)
    factor = ty / jnp.maximum(nu, EPS)
    factor = jnp.where(ty > MAX_NORM,
                       MAX_NORM / jnp.maximum(nu, EPS), factor)
    out_ref[...] = agg * (factor / deg)


def _tc_out(al, ar, dega, degb):
    return pl.pallas_call(
        _tc_out_body,
        grid=(GRID,),
        in_specs=[pl.BlockSpec((BN, HALF), lambda i: (i, 0)),
                  pl.BlockSpec((BN, HALF), lambda i: (i, 0)),
                  pl.BlockSpec((BN, 16), lambda i: (i, 0)),
                  pl.BlockSpec((BN, 16), lambda i: (i, 0))],
        out_specs=pl.BlockSpec((BN, D), lambda i: (i, 0)),
        out_shape=jax.ShapeDtypeStruct((N_NODES, D), jnp.float32),
    )(al, ar, dega, degb)


def _init_slab(tid, z_hbm, sh):
    for s in range(RPT // ZR):
        pltpu.sync_copy(z_hbm, sh.at[pl.ds(tid * RPT + s * ZR, ZR)])

    @pl.when(tid == 0)
    def _():
        pltpu.sync_copy(z_hbm.at[pl.ds(0, TAIL)], sh.at[pl.ds(TAIL0, TAIL)])


def _copy_out_slab(tid, sh, out, base):
    pltpu.sync_copy(sh.at[pl.ds(tid * RPT, RPT)],
                    out.at[pl.ds(base + tid * RPT, RPT)])

    @pl.when(tid == 0)
    def _():
        pltpu.sync_copy(sh.at[pl.ds(TAIL0, TAIL)],
                        out.at[pl.ds(base + TAIL0, TAIL)])


@functools.cache
def _sc_mesh():
    return plsc.VectorSubcoreMesh(core_axis_name="c", subcore_axis_name="s")


@functools.cache
def _sc_agg_kernel():
    return functools.partial(
        pl.kernel, mesh=_sc_mesh(),
        compiler_params=pltpu.CompilerParams(use_tc_tiling_on_sc=False),
        out_type=(jax.ShapeDtypeStruct((N_NODES, NC * QW), jnp.float32),
                  jax.ShapeDtypeStruct((N_NODES, NC * QW), jnp.float32)),
        scratch_types=[
            pltpu.VMEM((NCHUNK, K), jnp.int32),
            pltpu.VMEM((NCHUNK, K), jnp.int32),
            pltpu.VMEM((NBUF, K, QW), jnp.float32),
            pltpu.VMEM_SHARED((N_NODES, QW), jnp.float32),
            pltpu.SemaphoreType.DMA((NBUF,)),
            pltpu.SemaphoreType.DMA((NBUF,)),
        ])(_sc_agg_body)


def _sc_agg(hl2, hr2, src_all, dst_all, z64):
    return _sc_agg_kernel()(hl2, hr2, src_all, dst_all, z64)


def _sc_agg_body(hl_hbm, hr_hbm, src_hbm, dst_hbm, z64_hbm,
                 aggl_out, aggr_out,
                 src_v, dst_v, rows, agg_sh, gsem, ssem):
    cid = lax.axis_index("c")
    tid = lax.axis_index("s")
    wid = cid * NS + tid
    pltpu.sync_copy(src_hbm.at[wid], src_v)
    pltpu.sync_copy(dst_hbm.at[tid], dst_v)
    _init_slab(tid, z64_hbm, agg_sh)

    def copy_out(out):
        pltpu.sync_copy(agg_sh.at[pl.ds(tid * RPT, RPT)],
                        out.at[pl.ds(tid * RPT, RPT), pl.ds(QW * cid, QW)])

        @pl.when(tid == 0)
        def _():
            pltpu.sync_copy(agg_sh.at[pl.ds(TAIL0, TAIL)],
                            out.at[pl.ds(TAIL0, TAIL), pl.ds(QW * cid, QW)])

    def gather_from(h_hbm, j, b):
        pltpu.async_copy(h_hbm.at[src_v.at[j]], rows.at[b], gsem.at[b])

    def prologue(h_hbm):
        for b in range(NBUF):
            gather_from(h_hbm, b, b)

    def phase(h_hbm, out):
        plsc.subcore_barrier()

        def gather(j, b):
            gather_from(h_hbm, j, b)

        def wait_gather(b):
            pltpu.make_async_copy(h_hbm.at[pl.ds(0, K)], rows.at[b],
                                  gsem.at[b]).wait()

        def wait_scatter(b):
            pltpu.make_async_copy(rows.at[b], agg_sh.at[pl.ds(0, K)],
                                  ssem.at[b]).wait()

        def group_body(g, carry):
            for b in range(NBUF):
                j = g * NBUF + b
                wait_gather(b)
                pltpu.async_copy(rows.at[b], agg_sh.at[dst_v.at[j]],
                                 ssem.at[b], add=True)

                @pl.when(g < NGRP - 1)
                def _():
                    wait_scatter(b)
                    gather(j + NBUF, b)

            return carry

        lax.fori_loop(0, NGRP, group_body, 0)
        for b in range(NBUF):
            wait_scatter(b)

        plsc.subcore_barrier()

    prologue(hl_hbm)
    phase(hl_hbm, aggl_out)
    prologue(hr_hbm)
    copy_out(aggl_out)
    _init_slab(tid, z64_hbm, agg_sh)
    phase(hr_hbm, aggr_out)
    copy_out(aggr_out)


@functools.cache
def _sc_deg_kernel():
    return functools.partial(
        pl.kernel, mesh=_sc_mesh(),
        compiler_params=pltpu.CompilerParams(use_tc_tiling_on_sc=False),
        out_type=(jax.ShapeDtypeStruct((N_NODES, 16), jnp.float32),
                  jax.ShapeDtypeStruct((N_NODES, 16), jnp.float32)),
        scratch_types=[
            pltpu.VMEM((DNCHUNK, DK), jnp.int32),
            pltpu.VMEM((DK, 16), jnp.float32),
            pltpu.VMEM_SHARED((N_NODES, 16), jnp.float32),
        ])(_sc_deg_body)


def _sc_deg(dstd, z16, ones16):
    return _sc_deg_kernel()(dstd, z16, ones16)


def _sc_deg_body(dst_hbm, z16_hbm, ones_hbm,
                 dega_out, degb_out,
                 dst_v, ones_v, deg_sh):
    cid = lax.axis_index("c")
    tid = lax.axis_index("s")
    wid = cid * NS + tid
    pltpu.sync_copy(dst_hbm.at[wid], dst_v)
    pltpu.sync_copy(ones_hbm, ones_v)
    _init_slab(tid, z16_hbm, deg_sh)
    plsc.subcore_barrier()

    def body(j, carry):
        pltpu.sync_copy(ones_v, deg_sh.at[dst_v.at[j]], add=True)
        return carry

    lax.fori_loop(0, DNCHUNK, body, 0)

    plsc.subcore_barrier()

    @pl.when(cid == 0)
    def _():
        _copy_out_slab(tid, deg_sh, dega_out, 0)

    @pl.when(cid == 1)
    def _():
        _copy_out_slab(tid, deg_sh, degb_out, 0)


def kernel(x, edge_index, W0, b0, W1, b1, W2, b2):
    src = edge_index[0].astype(jnp.int32)
    dst = edge_index[1].astype(jnp.int32)
    src2 = (2 * src).reshape(1, NS, NCHUNK, K)
    src_all = jnp.concatenate([src2, src2 + 1], axis=0)
    src_all = src_all.reshape(NC * NS, NCHUNK, K)
    dst_all = dst.reshape(NS, NCHUNK, K)
    dst_deg = dst.reshape(NC * NS, DNCHUNK, DK)
    z64 = jnp.zeros((ZR, QW), jnp.float32)
    z16 = jnp.zeros((ZR, 16), jnp.float32)
    ones16 = jnp.ones((DK, 16), jnp.float32)

    def agg_layer(tl, tr):
        return _sc_agg(tl.reshape(NC * N_NODES, QW),
                       tr.reshape(NC * N_NODES, QW), src_all, dst_all, z64)

    dega, degb = _sc_deg(dst_deg, z16, ones16)
    tl, tr = _tc_pre(x, W0, b0.reshape(1, D))
    al, ar = agg_layer(tl, tr)
    tl, tr = _tc_mid_noact(al, ar, dega, degb, W1, b1.reshape(1, D))
    al, ar = agg_layer(tl, tr)
    tl, tr = _tc_mid_act(al, ar, dega, degb, W2, b2.reshape(1, D))
    al, ar = agg_layer(tl, tr)
    return _tc_out(al, ar, dega, degb)
